# baseline (device time: 195622 ns/iter reference)
import jax
import jax.numpy as jnp
from jax import lax
from jax.experimental import pallas as pl
from jax.experimental.pallas import tpu as pltpu

B, H, W, C = 2, 256, 256, 128
GH, GW = 512, 512
EPS = 1e-5

BR = 16
SNB = H // BR
NB = H // BR

_CP = getattr(pltpu, "CompilerParams", None) or pltpu.TPUCompilerParams
_ANY = pl.ANY
_MESH = pl.DeviceIdType.MESH


def _fused(x, k, Wp, bots):
    def body(xs_ref, xm_ref, xa_ref, k_ref, wp_ref, bots_ref, o_ref,
             col_halo, row_halo, send_col, send_row, ext_row,
             acc, sbuf, rxbuf, sbuf2, rybuf, stats_s, carry,
             local_sems, halo_send, halo_recv, stat_send, stat_recv):
        i = pl.program_id(0)
        mx = lax.axis_index("x")
        my = lax.axis_index("y")
        cy = jnp.where(my == 0, W - 1, 0)
        rx = jnp.where(mx == 0, H - 1, 0)

        def col_rdma():
            return pltpu.make_async_remote_copy(
                src_ref=send_col, dst_ref=col_halo,
                send_sem=halo_send.at[0], recv_sem=halo_recv.at[0],
                device_id=(mx, 1 - my), device_id_type=_MESH)

        def row_rdma():
            return pltpu.make_async_remote_copy(
                src_ref=ext_row, dst_ref=row_halo,
                send_sem=halo_send.at[1], recv_sem=halo_recv.at[1],
                device_id=(1 - mx, my), device_id_type=_MESH)

        def loc_col():
            return pltpu.make_async_copy(
                xa_ref.at[:, :, pl.ds(cy, 1), :], send_col, local_sems.at[0])

        def loc_row():
            return pltpu.make_async_copy(
                xa_ref.at[:, pl.ds(rx, 1), :, :], send_row, local_sems.at[1])

        @pl.when(i == 0)
        def _():
            bar = pltpu.get_barrier_semaphore()
            pl.semaphore_signal(bar, inc=1, device_id=(1 - mx, my),
                                device_id_type=_MESH)
            pl.semaphore_signal(bar, inc=1, device_id=(mx, 1 - my),
                                device_id_type=_MESH)
            pl.semaphore_wait(bar, 2)
            loc_col().start()
            loc_row().start()
            loc_col().wait()
            col_rdma().start()
            acc[...] = jnp.zeros_like(acc)

        @pl.when(i == 1)
        def _():
            loc_row().wait()
            col_rdma().wait()
            rowv = send_row[...]
            colr = col_halo[:, pl.ds(rx, 1), :, :]
            left = jnp.where(my == 0, rowv[:, :, 0:1, :], colr)
            right = jnp.where(my == 0, colr, rowv[:, :, W - 1:W, :])
            ext_row[...] = jnp.concatenate([left, rowv, right], axis=2)
            row_rdma().start()

        @pl.when(i == 2)
        def _():
            row_rdma().wait()

        @pl.when(i < SNB)
        def _():
            xb = xs_ref[...]
            s = jnp.sum(xb, axis=(1, 2))
            s2 = jnp.sum(xb * xb, axis=(1, 2))
            acc[...] = acc[...] + jnp.stack([s, s2], axis=0)

        @pl.when(i == SNB - 1)
        def _():
            sbuf[...] = acc[...]
            rdx = pltpu.make_async_remote_copy(
                src_ref=sbuf, dst_ref=rxbuf,
                send_sem=stat_send.at[0], recv_sem=stat_recv.at[0],
                device_id=(1 - mx, my), device_id_type=_MESH)
            rdx.start()
            rdx.wait()
            sbuf2[...] = sbuf[...] + rxbuf[...]
            rdy = pltpu.make_async_remote_copy(
                src_ref=sbuf2, dst_ref=rybuf,
                send_sem=stat_send.at[1], recv_sem=stat_recv.at[1],
                device_id=(mx, 1 - my), device_id_type=_MESH)
            rdy.start()
            rdy.wait()
            tot = sbuf2[...] + rybuf[...]
            n = float(GH * GW)
            mean = tot[0] / n
            var = tot[1] / n - mean * mean
            stats_s[...] = jnp.stack([mean, lax.rsqrt(var + EPS)], axis=0)

        @pl.when(i >= SNB)
        def _():
            j = i - SNB
            st = stats_s[...]
            mean = st[0]
            rstd = st[1]
            mb = mean[:, None, None, :]
            rb = rstd[:, None, None, :]

            xb = xm_ref[...]
            ch_blk = col_halo[:, pl.ds(j * BR, BR), :, :]
            lc = jnp.where(my == 0, xb[:, :, 0:1, :], ch_blk)
            rc = jnp.where(my == 0, ch_blk, xb[:, :, W - 1:W, :])
            hW = (jnp.concatenate([lc, xb, rc], axis=2) - mb) * rb

            row0 = xb[:, 0, :, :]
            ch0 = col_halo[:, 0, :, :]
            e_l = jnp.where(my == 0, row0[:, 0:1, :], ch0)
            e_r = jnp.where(my == 0, ch0, row0[:, W - 1:W, :])
            edge_top = jnp.concatenate([e_l, row0, e_r], axis=1)
            top_raw = jnp.where(
                j == 0,
                jnp.where(mx == 1, row_halo[:, 0], edge_top),
                carry[:, 0])
            top_n = (top_raw - mean[:, None, :]) * rstd[:, None, :]

            rb_idx = jnp.minimum((j + 1) * BR, H - 1)
            bot256 = bots_ref[0]
            ch_b = col_halo[:, pl.ds(rb_idx, 1), 0, :]
            b_l = jnp.where(my == 0, bot256[:, 0:1, :], ch_b)
            b_r = jnp.where(my == 0, ch_b, bot256[:, W - 1:W, :])
            bot_ext = jnp.concatenate([b_l, bot256, b_r], axis=1)
            bot_raw = jnp.where((j == NB - 1) & (mx == 0),
                                row_halo[:, 0], bot_ext)
            bot_n = (bot_raw - mean[:, None, :]) * rstd[:, None, :]

            padded = jnp.concatenate(
                [top_n[:, None], hW, bot_n[:, None]], axis=1)

            kv = k_ref[...]
            conv = jnp.zeros_like(xb)
            for di in range(3):
                for dj in range(3):
                    conv = conv + (padded[:, di:di + BR, dj:dj + W, :]
                                   * kv[di, dj][None, None, None, :])
            a = conv * jax.nn.sigmoid(conv)
            o = jnp.dot(a.reshape(B * BR * W, C), wp_ref[...],
                        preferred_element_type=jnp.float32)
            o_ref[...] = xb + o.reshape(B, BR, W, C)

            lastr = xb[:, BR - 1, :, :]
            chl = col_halo[:, pl.ds(j * BR + BR - 1, 1), 0, :]
            c_l = jnp.where(my == 0, lastr[:, 0:1, :], chl)
            c_r = jnp.where(my == 0, chl, lastr[:, W - 1:W, :])
            carry[...] = jnp.concatenate([c_l, lastr, c_r], axis=1)[:, None]

    grid = SNB + NB
    return pl.pallas_call(
        body,
        grid=(grid,),
        out_shape=jax.ShapeDtypeStruct((B, H, W, C), jnp.float32),
        in_specs=[
            pl.BlockSpec((B, BR, W, C),
                         lambda i: (0, jnp.minimum(i, SNB - 1), 0, 0)),
            pl.BlockSpec((B, BR, W, C),
                         lambda i: (0, jnp.maximum(i - SNB, 0), 0, 0)),
            pl.BlockSpec(memory_space=_ANY),
            pl.BlockSpec((3, 3, C), lambda i: (0, 0, 0)),
            pl.BlockSpec((C, C), lambda i: (0, 0)),
            pl.BlockSpec((1, B, W, C),
                         lambda i: (jnp.maximum(i - SNB, 0), 0, 0, 0)),
        ],
        out_specs=pl.BlockSpec((B, BR, W, C),
                               lambda i: (0, jnp.maximum(i - SNB, 0), 0, 0)),
        scratch_shapes=[
            pltpu.VMEM((B, H, 1, C), jnp.float32),
            pltpu.VMEM((B, 1, W + 2, C), jnp.float32),
            pltpu.VMEM((B, H, 1, C), jnp.float32),
            pltpu.VMEM((B, 1, W, C), jnp.float32),
            pltpu.VMEM((B, 1, W + 2, C), jnp.float32),
            pltpu.VMEM((2, B, C), jnp.float32),
            pltpu.VMEM((2, B, C), jnp.float32),
            pltpu.VMEM((2, B, C), jnp.float32),
            pltpu.VMEM((2, B, C), jnp.float32),
            pltpu.VMEM((2, B, C), jnp.float32),
            pltpu.VMEM((2, B, C), jnp.float32),
            pltpu.VMEM((B, 1, W + 2, C), jnp.float32),
            pltpu.SemaphoreType.DMA((2,)),
            pltpu.SemaphoreType.DMA((2,)),
            pltpu.SemaphoreType.DMA((2,)),
            pltpu.SemaphoreType.DMA((2,)),
            pltpu.SemaphoreType.DMA((2,)),
        ],
        compiler_params=_CP(
            collective_id=0, dimension_semantics=("arbitrary",)),
    )(x, x, x, k, Wp, bots)


def kernel(x, k, Wp):
    bot_idx = tuple(min((i + 1) * BR, H - 1) for i in range(NB))
    bots = jnp.moveaxis(x[:, bot_idx, :, :], 1, 0)
    return _fused(x, k, Wp, bots)


# device time: 192752 ns/iter; 1.0149x vs baseline; 1.0149x over previous
import jax
import jax.numpy as jnp
from jax import lax
from jax.experimental import pallas as pl
from jax.experimental.pallas import tpu as pltpu

B, H, W, C = 2, 256, 256, 128
GH, GW = 512, 512
EPS = 1e-5

BR = 16
SNB = H // BR
NB = H // BR

_CP = getattr(pltpu, "CompilerParams", None) or pltpu.TPUCompilerParams
_MESH = pl.DeviceIdType.MESH


def _fused(x, k, Wp, bots):
    def body(x_ref, k_ref, wp_ref, bots_ref, o_ref,
             col_halo, row_halo, send_col, send_row, ext_row,
             acc, sbuf, rxbuf, sbuf2, rybuf, stats_s, carry,
             halo_send, halo_recv, stat_send, stat_recv):
        i = pl.program_id(0)
        mx = lax.axis_index("x")
        my = lax.axis_index("y")
        cy = jnp.where(my == 0, W - 1, 0)
        rx = jnp.where(mx == 0, H - 1, 0)

        def col_rdma():
            return pltpu.make_async_remote_copy(
                src_ref=send_col, dst_ref=col_halo,
                send_sem=halo_send.at[0], recv_sem=halo_recv.at[0],
                device_id=(mx, 1 - my), device_id_type=_MESH)

        def row_rdma():
            return pltpu.make_async_remote_copy(
                src_ref=ext_row, dst_ref=row_halo,
                send_sem=halo_send.at[1], recv_sem=halo_recv.at[1],
                device_id=(1 - mx, my), device_id_type=_MESH)

        @pl.when(i == 0)
        def _():
            bar = pltpu.get_barrier_semaphore()
            pl.semaphore_signal(bar, inc=1, device_id=(1 - mx, my),
                                device_id_type=_MESH)
            pl.semaphore_signal(bar, inc=1, device_id=(mx, 1 - my),
                                device_id_type=_MESH)
            pl.semaphore_wait(bar, 2)
            acc[...] = jnp.zeros_like(acc)

        @pl.when(i < SNB)
        def _():
            xb = x_ref[...]
            s = jnp.sum(xb, axis=(1, 2))
            s2 = jnp.sum(xb * xb, axis=(1, 2))
            acc[...] = acc[...] + jnp.stack([s, s2], axis=0)
            send_col[:, pl.ds(i * BR, BR), :, :] = x_ref[:, :, pl.ds(cy, 1), :]
            @pl.when(i == jnp.where(mx == 0, SNB - 1, 0))
            def _():
                send_row[...] = x_ref[
                    :, pl.ds(jnp.where(mx == 0, BR - 1, 0), 1), :, :]

        @pl.when(i == SNB - 1)
        def _():
            col_rdma().start()
            sbuf[...] = acc[...]
            rdx = pltpu.make_async_remote_copy(
                src_ref=sbuf, dst_ref=rxbuf,
                send_sem=stat_send.at[0], recv_sem=stat_recv.at[0],
                device_id=(1 - mx, my), device_id_type=_MESH)
            rdx.start()
            rdx.wait()
            sbuf2[...] = sbuf[...] + rxbuf[...]
            rdy = pltpu.make_async_remote_copy(
                src_ref=sbuf2, dst_ref=rybuf,
                send_sem=stat_send.at[1], recv_sem=stat_recv.at[1],
                device_id=(mx, 1 - my), device_id_type=_MESH)
            rdy.start()
            rdy.wait()
            tot = sbuf2[...] + rybuf[...]
            n = float(GH * GW)
            mean = tot[0] / n
            var = tot[1] / n - mean * mean
            stats_s[...] = jnp.stack([mean, lax.rsqrt(var + EPS)], axis=0)
            col_rdma().wait()
            rowv = send_row[...]
            colr = col_halo[:, pl.ds(rx, 1), :, :]
            left = jnp.where(my == 0, rowv[:, :, 0:1, :], colr)
            right = jnp.where(my == 0, colr, rowv[:, :, W - 1:W, :])
            ext_row[...] = jnp.concatenate([left, rowv, right], axis=2)
            row_rdma().start()

        @pl.when(i == SNB)
        def _():
            row_rdma().wait()

        @pl.when(i >= SNB)
        def _():
            j = i - SNB
            st = stats_s[...]
            mean = st[0]
            rstd = st[1]
            mb = mean[:, None, None, :]
            rb = rstd[:, None, None, :]

            xb = x_ref[...]
            ch_blk = col_halo[:, pl.ds(j * BR, BR), :, :]
            lc = jnp.where(my == 0, xb[:, :, 0:1, :], ch_blk)
            rc = jnp.where(my == 0, ch_blk, xb[:, :, W - 1:W, :])
            hW = (jnp.concatenate([lc, xb, rc], axis=2) - mb) * rb

            row0 = xb[:, 0, :, :]
            ch0 = col_halo[:, 0, :, :]
            e_l = jnp.where(my == 0, row0[:, 0:1, :], ch0)
            e_r = jnp.where(my == 0, ch0, row0[:, W - 1:W, :])
            edge_top = jnp.concatenate([e_l, row0, e_r], axis=1)
            top_raw = jnp.where(
                j == 0,
                jnp.where(mx == 1, row_halo[:, 0], edge_top),
                carry[:, 0])
            top_n = (top_raw - mean[:, None, :]) * rstd[:, None, :]

            rb_idx = jnp.minimum((j + 1) * BR, H - 1)
            bot256 = bots_ref[0]
            ch_b = col_halo[:, pl.ds(rb_idx, 1), 0, :]
            b_l = jnp.where(my == 0, bot256[:, 0:1, :], ch_b)
            b_r = jnp.where(my == 0, ch_b, bot256[:, W - 1:W, :])
            bot_ext = jnp.concatenate([b_l, bot256, b_r], axis=1)
            bot_raw = jnp.where((j == NB - 1) & (mx == 0),
                                row_halo[:, 0], bot_ext)
            bot_n = (bot_raw - mean[:, None, :]) * rstd[:, None, :]

            padded = jnp.concatenate(
                [top_n[:, None], hW, bot_n[:, None]], axis=1)

            kv = k_ref[...]
            conv = jnp.zeros_like(xb)
            for di in range(3):
                for dj in range(3):
                    conv = conv + (padded[:, di:di + BR, dj:dj + W, :]
                                   * kv[di, dj][None, None, None, :])
            a = conv * jax.nn.sigmoid(conv)
            o = jnp.dot(a.reshape(B * BR * W, C), wp_ref[...],
                        preferred_element_type=jnp.float32)
            o_ref[...] = xb + o.reshape(B, BR, W, C)

            lastr = xb[:, BR - 1, :, :]
            chl = col_halo[:, pl.ds(j * BR + BR - 1, 1), 0, :]
            c_l = jnp.where(my == 0, lastr[:, 0:1, :], chl)
            c_r = jnp.where(my == 0, chl, lastr[:, W - 1:W, :])
            carry[...] = jnp.concatenate([c_l, lastr, c_r], axis=1)[:, None]

    grid = SNB + NB
    return pl.pallas_call(
        body,
        grid=(grid,),
        out_shape=jax.ShapeDtypeStruct((B, H, W, C), jnp.float32),
        in_specs=[
            pl.BlockSpec((B, BR, W, C),
                         lambda i: (0, jnp.where(i < SNB, i, i - SNB), 0, 0)),
            pl.BlockSpec((3, 3, C), lambda i: (0, 0, 0)),
            pl.BlockSpec((C, C), lambda i: (0, 0)),
            pl.BlockSpec((1, B, W, C),
                         lambda i: (jnp.maximum(i - SNB, 0), 0, 0, 0)),
        ],
        out_specs=pl.BlockSpec((B, BR, W, C),
                               lambda i: (0, jnp.maximum(i - SNB, 0), 0, 0)),
        scratch_shapes=[
            pltpu.VMEM((B, H, 1, C), jnp.float32),
            pltpu.VMEM((B, 1, W + 2, C), jnp.float32),
            pltpu.VMEM((B, H, 1, C), jnp.float32),
            pltpu.VMEM((B, 1, W, C), jnp.float32),
            pltpu.VMEM((B, 1, W + 2, C), jnp.float32),
            pltpu.VMEM((2, B, C), jnp.float32),
            pltpu.VMEM((2, B, C), jnp.float32),
            pltpu.VMEM((2, B, C), jnp.float32),
            pltpu.VMEM((2, B, C), jnp.float32),
            pltpu.VMEM((2, B, C), jnp.float32),
            pltpu.VMEM((2, B, C), jnp.float32),
            pltpu.VMEM((B, 1, W + 2, C), jnp.float32),
            pltpu.SemaphoreType.DMA((2,)),
            pltpu.SemaphoreType.DMA((2,)),
            pltpu.SemaphoreType.DMA((2,)),
            pltpu.SemaphoreType.DMA((2,)),
        ],
        compiler_params=_CP(
            collective_id=0, dimension_semantics=("arbitrary",)),
    )(x, k, Wp, bots)


def kernel(x, k, Wp):
    bot_idx = tuple(min((i + 1) * BR, H - 1) for i in range(NB))
    bots = jnp.moveaxis(x[:, bot_idx, :, :], 1, 0)
    return _fused(x, k, Wp, bots)


# device time: 146874 ns/iter; 1.3319x vs baseline; 1.3124x over previous
import jax
import jax.numpy as jnp
from jax import lax
from jax.experimental import pallas as pl
from jax.experimental.pallas import tpu as pltpu

B, H, W, C = 2, 256, 256, 128
GH, GW = 512, 512
EPS = 1e-5

SBR = 32
SNB = H // SBR
BR = 16
NB = H // BR

_CP = getattr(pltpu, "CompilerParams", None) or pltpu.TPUCompilerParams
_MESH = pl.DeviceIdType.MESH


def _prep(x):
    def body(x_ref, st_out, col_out, row_out,
             send_col, send_row, ext_row,
             acc, sbuf, rxbuf, sbuf2, rybuf,
             halo_send, halo_recv, stat_send, stat_recv):
        i = pl.program_id(0)
        mx = lax.axis_index("x")
        my = lax.axis_index("y")
        cy = jnp.where(my == 0, W - 1, 0)
        rx = jnp.where(mx == 0, H - 1, 0)

        @pl.when(i == 0)
        def _():
            bar = pltpu.get_barrier_semaphore()
            pl.semaphore_signal(bar, inc=1, device_id=(1 - mx, my),
                                device_id_type=_MESH)
            pl.semaphore_signal(bar, inc=1, device_id=(mx, 1 - my),
                                device_id_type=_MESH)
            pl.semaphore_wait(bar, 2)
            acc[...] = jnp.zeros_like(acc)

        xb = x_ref[...]
        s = jnp.sum(xb, axis=(1, 2))
        s2 = jnp.sum(xb * xb, axis=(1, 2))
        acc[...] = acc[...] + jnp.stack([s, s2], axis=0)
        send_col[:, pl.ds(i * SBR, SBR), :, :] = x_ref[:, :, pl.ds(cy, 1), :]

        @pl.when(i == jnp.where(mx == 0, SNB - 1, 0))
        def _():
            send_row[...] = x_ref[
                :, pl.ds(jnp.where(mx == 0, SBR - 1, 0), 1), :, :]

        @pl.when(i == SNB - 1)
        def _():
            colx = pltpu.make_async_remote_copy(
                src_ref=send_col, dst_ref=col_out,
                send_sem=halo_send.at[0], recv_sem=halo_recv.at[0],
                device_id=(mx, 1 - my), device_id_type=_MESH)
            colx.start()
            sbuf[...] = acc[...]
            rdx = pltpu.make_async_remote_copy(
                src_ref=sbuf, dst_ref=rxbuf,
                send_sem=stat_send.at[0], recv_sem=stat_recv.at[0],
                device_id=(1 - mx, my), device_id_type=_MESH)
            rdx.start()
            rdx.wait()
            sbuf2[...] = sbuf[...] + rxbuf[...]
            rdy = pltpu.make_async_remote_copy(
                src_ref=sbuf2, dst_ref=rybuf,
                send_sem=stat_send.at[1], recv_sem=stat_recv.at[1],
                device_id=(mx, 1 - my), device_id_type=_MESH)
            rdy.start()
            rdy.wait()
            tot = sbuf2[...] + rybuf[...]
            n = float(GH * GW)
            mean = tot[0] / n
            var = tot[1] / n - mean * mean
            st_out[...] = jnp.stack([mean, lax.rsqrt(var + EPS)], axis=0)

            colx.wait()
            rowv = send_row[...]
            colr = col_out[:, pl.ds(rx, 1), :, :]
            left = jnp.where(my == 0, rowv[:, :, 0:1, :], colr)
            right = jnp.where(my == 0, colr, rowv[:, :, W - 1:W, :])
            ext_row[...] = jnp.concatenate([left, rowv, right], axis=2)
            rowx = pltpu.make_async_remote_copy(
                src_ref=ext_row, dst_ref=row_out,
                send_sem=halo_send.at[1], recv_sem=halo_recv.at[1],
                device_id=(1 - mx, my), device_id_type=_MESH)
            rowx.start()
            rowx.wait()

    return pl.pallas_call(
        body,
        grid=(SNB,),
        out_shape=(
            jax.ShapeDtypeStruct((2, B, C), jnp.float32),
            jax.ShapeDtypeStruct((B, H, 1, C), jnp.float32),
            jax.ShapeDtypeStruct((B, 1, W + 2, C), jnp.float32),
        ),
        in_specs=[pl.BlockSpec((B, SBR, W, C), lambda i: (0, i, 0, 0))],
        out_specs=(
            pl.BlockSpec((2, B, C), lambda i: (0, 0, 0)),
            pl.BlockSpec((B, H, 1, C), lambda i: (0, 0, 0, 0)),
            pl.BlockSpec((B, 1, W + 2, C), lambda i: (0, 0, 0, 0)),
        ),
        scratch_shapes=[
            pltpu.VMEM((B, H, 1, C), jnp.float32),
            pltpu.VMEM((B, 1, W, C), jnp.float32),
            pltpu.VMEM((B, 1, W + 2, C), jnp.float32),
            pltpu.VMEM((2, B, C), jnp.float32),
            pltpu.VMEM((2, B, C), jnp.float32),
            pltpu.VMEM((2, B, C), jnp.float32),
            pltpu.VMEM((2, B, C), jnp.float32),
            pltpu.VMEM((2, B, C), jnp.float32),
            pltpu.SemaphoreType.DMA((2,)),
            pltpu.SemaphoreType.DMA((2,)),
            pltpu.SemaphoreType.DMA((2,)),
            pltpu.SemaphoreType.DMA((2,)),
        ],
        compiler_params=_CP(
            collective_id=0, dimension_semantics=("arbitrary",)),
    )(x)


def _main(x, k, Wp, stats, col_halo, row_halo, bots):
    def body(x_ref, k_ref, wp_ref, st_ref, col_ref, row_ref, bots_ref,
             o_ref, carry):
        j = pl.program_id(0)
        mx = lax.axis_index("x")
        my = lax.axis_index("y")
        st = st_ref[...]
        mean = st[0]
        rstd = st[1]
        mb = mean[:, None, None, :]
        rb = rstd[:, None, None, :]

        xb = x_ref[...]
        ch_blk = col_ref[:, pl.ds(j * BR, BR), :, :]
        lc = jnp.where(my == 0, xb[:, :, 0:1, :], ch_blk)
        rc = jnp.where(my == 0, ch_blk, xb[:, :, W - 1:W, :])
        hW = (jnp.concatenate([lc, xb, rc], axis=2) - mb) * rb

        row0 = xb[:, 0, :, :]
        ch0 = col_ref[:, 0, :, :]
        e_l = jnp.where(my == 0, row0[:, 0:1, :], ch0)
        e_r = jnp.where(my == 0, ch0, row0[:, W - 1:W, :])
        edge_top = jnp.concatenate([e_l, row0, e_r], axis=1)
        top_raw = jnp.where(
            j == 0,
            jnp.where(mx == 1, row_ref[:, 0], edge_top),
            carry[:, 0])
        top_n = (top_raw - mean[:, None, :]) * rstd[:, None, :]

        rb_idx = jnp.minimum((j + 1) * BR, H - 1)
        bot256 = bots_ref[0]
        ch_b = col_ref[:, pl.ds(rb_idx, 1), 0, :]
        b_l = jnp.where(my == 0, bot256[:, 0:1, :], ch_b)
        b_r = jnp.where(my == 0, ch_b, bot256[:, W - 1:W, :])
        bot_ext = jnp.concatenate([b_l, bot256, b_r], axis=1)
        bot_raw = jnp.where((j == NB - 1) & (mx == 0),
                            row_ref[:, 0], bot_ext)
        bot_n = (bot_raw - mean[:, None, :]) * rstd[:, None, :]

        padded = jnp.concatenate(
            [top_n[:, None], hW, bot_n[:, None]], axis=1)

        kv = k_ref[...]
        conv = jnp.zeros_like(xb)
        for di in range(3):
            for dj in range(3):
                conv = conv + (padded[:, di:di + BR, dj:dj + W, :]
                               * kv[di, dj][None, None, None, :])
        a = conv * jax.nn.sigmoid(conv)
        o = jnp.dot(a.reshape(B * BR * W, C), wp_ref[...],
                    preferred_element_type=jnp.float32)
        o_ref[...] = xb + o.reshape(B, BR, W, C)

        lastr = xb[:, BR - 1, :, :]
        chl = col_ref[:, pl.ds(j * BR + BR - 1, 1), 0, :]
        c_l = jnp.where(my == 0, lastr[:, 0:1, :], chl)
        c_r = jnp.where(my == 0, chl, lastr[:, W - 1:W, :])
        carry[...] = jnp.concatenate([c_l, lastr, c_r], axis=1)[:, None]

    return pl.pallas_call(
        body,
        grid=(NB,),
        out_shape=jax.ShapeDtypeStruct((B, H, W, C), jnp.float32),
        in_specs=[
            pl.BlockSpec((B, BR, W, C), lambda i: (0, i, 0, 0)),
            pl.BlockSpec((3, 3, C), lambda i: (0, 0, 0)),
            pl.BlockSpec((C, C), lambda i: (0, 0)),
            pl.BlockSpec((2, B, C), lambda i: (0, 0, 0)),
            pl.BlockSpec((B, H, 1, C), lambda i: (0, 0, 0, 0)),
            pl.BlockSpec((B, 1, W + 2, C), lambda i: (0, 0, 0, 0)),
            pl.BlockSpec((1, B, W, C), lambda i: (i, 0, 0, 0)),
        ],
        out_specs=pl.BlockSpec((B, BR, W, C), lambda i: (0, i, 0, 0)),
        scratch_shapes=[
            pltpu.VMEM((B, 1, W + 2, C), jnp.float32),
        ],
        compiler_params=_CP(dimension_semantics=("arbitrary",)),
    )(x, k, Wp, stats, col_halo, row_halo, bots)


def kernel(x, k, Wp):
    stats, col_halo, row_halo = _prep(x)
    bot_idx = tuple(min((i + 1) * BR, H - 1) for i in range(NB))
    bots = jnp.moveaxis(x[:, bot_idx, :, :], 1, 0)
    return _main(x, k, Wp, stats, col_halo, row_halo, bots)


# device time: 125831 ns/iter; 1.5546x vs baseline; 1.1672x over previous
import jax
import jax.numpy as jnp
from jax import lax
from jax.experimental import pallas as pl
from jax.experimental.pallas import tpu as pltpu

B, H, W, C = 2, 256, 256, 128
GH, GW = 512, 512
EPS = 1e-5

SBR = 32
SNB = H // SBR
BR = 16
NB = H // BR

_CP = getattr(pltpu, "CompilerParams", None) or pltpu.TPUCompilerParams
_MESH = pl.DeviceIdType.MESH


def _prep(x):
    def body(x_ref, st_out, col_out, row_out,
             send_col, send_row, ext_row,
             acc, sbuf, rxbuf, sbuf2, rybuf,
             halo_send, halo_recv, stat_send, stat_recv):
        i = pl.program_id(0)
        mx = lax.axis_index("x")
        my = lax.axis_index("y")
        cy = jnp.where(my == 0, W - 1, 0)
        rx = jnp.where(mx == 0, H - 1, 0)

        @pl.when(i == 0)
        def _():
            bar = pltpu.get_barrier_semaphore()
            pl.semaphore_signal(bar, inc=1, device_id=(1 - mx, my),
                                device_id_type=_MESH)
            pl.semaphore_signal(bar, inc=1, device_id=(mx, 1 - my),
                                device_id_type=_MESH)
            pl.semaphore_wait(bar, 2)
            acc[...] = jnp.zeros_like(acc)

        xb = x_ref[...]
        s = jnp.sum(xb, axis=1)
        s2 = jnp.sum(xb * xb, axis=1)
        acc[...] = acc[...] + jnp.stack([s, s2], axis=0)
        send_col[:, pl.ds(i * SBR, SBR), :, :] = x_ref[:, :, pl.ds(cy, 1), :]

        @pl.when(i == jnp.where(mx == 0, SNB - 1, 0))
        def _():
            send_row[...] = x_ref[
                :, pl.ds(jnp.where(mx == 0, SBR - 1, 0), 1), :, :]

        @pl.when(i == SNB - 1)
        def _():
            colx = pltpu.make_async_remote_copy(
                src_ref=send_col, dst_ref=col_out,
                send_sem=halo_send.at[0], recv_sem=halo_recv.at[0],
                device_id=(mx, 1 - my), device_id_type=_MESH)
            colx.start()
            sbuf[...] = jnp.sum(acc[...], axis=2)
            rdx = pltpu.make_async_remote_copy(
                src_ref=sbuf, dst_ref=rxbuf,
                send_sem=stat_send.at[0], recv_sem=stat_recv.at[0],
                device_id=(1 - mx, my), device_id_type=_MESH)
            rdx.start()
            rdx.wait()
            sbuf2[...] = sbuf[...] + rxbuf[...]
            rdy = pltpu.make_async_remote_copy(
                src_ref=sbuf2, dst_ref=rybuf,
                send_sem=stat_send.at[1], recv_sem=stat_recv.at[1],
                device_id=(mx, 1 - my), device_id_type=_MESH)
            rdy.start()
            rdy.wait()
            tot = sbuf2[...] + rybuf[...]
            n = float(GH * GW)
            mean = tot[0] / n
            var = tot[1] / n - mean * mean
            st_out[...] = jnp.stack([mean, lax.rsqrt(var + EPS)], axis=0)

            colx.wait()
            rowv = send_row[...]
            colr = col_out[:, pl.ds(rx, 1), :, :]
            left = jnp.where(my == 0, rowv[:, :, 0:1, :], colr)
            right = jnp.where(my == 0, colr, rowv[:, :, W - 1:W, :])
            ext_row[...] = jnp.concatenate([left, rowv, right], axis=2)
            rowx = pltpu.make_async_remote_copy(
                src_ref=ext_row, dst_ref=row_out,
                send_sem=halo_send.at[1], recv_sem=halo_recv.at[1],
                device_id=(1 - mx, my), device_id_type=_MESH)
            rowx.start()
            rowx.wait()

    return pl.pallas_call(
        body,
        grid=(SNB,),
        out_shape=(
            jax.ShapeDtypeStruct((2, B, C), jnp.float32),
            jax.ShapeDtypeStruct((B, H, 1, C), jnp.float32),
            jax.ShapeDtypeStruct((B, 1, W + 2, C), jnp.float32),
        ),
        in_specs=[pl.BlockSpec((B, SBR, W, C), lambda i: (0, i, 0, 0))],
        out_specs=(
            pl.BlockSpec((2, B, C), lambda i: (0, 0, 0)),
            pl.BlockSpec((B, H, 1, C), lambda i: (0, 0, 0, 0)),
            pl.BlockSpec((B, 1, W + 2, C), lambda i: (0, 0, 0, 0)),
        ),
        scratch_shapes=[
            pltpu.VMEM((B, H, 1, C), jnp.float32),
            pltpu.VMEM((B, 1, W, C), jnp.float32),
            pltpu.VMEM((B, 1, W + 2, C), jnp.float32),
            pltpu.VMEM((2, B, W, C), jnp.float32),
            pltpu.VMEM((2, B, C), jnp.float32),
            pltpu.VMEM((2, B, C), jnp.float32),
            pltpu.VMEM((2, B, C), jnp.float32),
            pltpu.VMEM((2, B, C), jnp.float32),
            pltpu.SemaphoreType.DMA((2,)),
            pltpu.SemaphoreType.DMA((2,)),
            pltpu.SemaphoreType.DMA((2,)),
            pltpu.SemaphoreType.DMA((2,)),
        ],
        compiler_params=_CP(
            collective_id=0, dimension_semantics=("arbitrary",),
            vmem_limit_bytes=56 * 1024 * 1024),
    )(x)


def _main(x, k, Wp, stats, col_halo, row_halo, bots):
    def body(x_ref, k_ref, wp_ref, st_ref, col_ref, row_ref, bots_ref,
             o_ref, carry, hpad):
        j = pl.program_id(0)
        mx = lax.axis_index("x")
        my = lax.axis_index("y")
        st = st_ref[...]
        mean = st[0]
        rstd = st[1]
        mb = mean[:, None, None, :]
        rb = rstd[:, None, None, :]

        xb = x_ref[...]
        ch_blk = col_ref[:, pl.ds(j * BR, BR), :, :]
        lc = jnp.where(my == 0, xb[:, :, 0:1, :], ch_blk)
        rc = jnp.where(my == 0, ch_blk, xb[:, :, W - 1:W, :])
        hpad[:, 1:BR + 1, 8:8 + W, :] = (xb - mb) * rb
        hpad[:, 1:BR + 1, 7:8, :] = (lc - mb) * rb
        hpad[:, 1:BR + 1, 8 + W:9 + W, :] = (rc - mb) * rb

        row0 = xb[:, 0, :, :]
        ch0 = col_ref[:, 0, :, :]
        e_l = jnp.where(my == 0, row0[:, 0:1, :], ch0)
        e_r = jnp.where(my == 0, ch0, row0[:, W - 1:W, :])
        edge_top = jnp.concatenate([e_l, row0, e_r], axis=1)
        top_raw = jnp.where(
            j == 0,
            jnp.where(mx == 1, row_ref[:, 0], edge_top),
            carry[:, 0])
        top_n = (top_raw - mean[:, None, :]) * rstd[:, None, :]

        rb_idx = jnp.minimum((j + 1) * BR, H - 1)
        bot256 = bots_ref[0]
        ch_b = col_ref[:, pl.ds(rb_idx, 1), 0, :]
        b_l = jnp.where(my == 0, bot256[:, 0:1, :], ch_b)
        b_r = jnp.where(my == 0, ch_b, bot256[:, W - 1:W, :])
        bot_ext = jnp.concatenate([b_l, bot256, b_r], axis=1)
        bot_raw = jnp.where((j == NB - 1) & (mx == 0),
                            row_ref[:, 0], bot_ext)
        bot_n = (bot_raw - mean[:, None, :]) * rstd[:, None, :]

        hpad[:, 0:1, 7:9 + W, :] = top_n[:, None]
        hpad[:, BR + 1:BR + 2, 7:9 + W, :] = bot_n[:, None]

        kv = k_ref[...]
        conv = None
        for dj in range(3):
            adj = (hpad[:, 0:BR, :, :] * kv[0, dj][None, None, None, :]
                   + hpad[:, 1:BR + 1, :, :] * kv[1, dj][None, None, None, :]
                   + hpad[:, 2:BR + 2, :, :] * kv[2, dj][None, None, None, :])
            sl = adj[:, :, 7 + dj:7 + dj + W, :]
            conv = sl if conv is None else conv + sl
        a = conv * jax.nn.sigmoid(conv)
        o = jnp.dot(a.reshape(B * BR * W, C), wp_ref[...],
                    preferred_element_type=jnp.float32)
        o_ref[...] = xb + o.reshape(B, BR, W, C)

        lastr = xb[:, BR - 1, :, :]
        chl = col_ref[:, pl.ds(j * BR + BR - 1, 1), 0, :]
        c_l = jnp.where(my == 0, lastr[:, 0:1, :], chl)
        c_r = jnp.where(my == 0, chl, lastr[:, W - 1:W, :])
        carry[...] = jnp.concatenate([c_l, lastr, c_r], axis=1)[:, None]

    return pl.pallas_call(
        body,
        grid=(NB,),
        out_shape=jax.ShapeDtypeStruct((B, H, W, C), jnp.float32),
        in_specs=[
            pl.BlockSpec((B, BR, W, C), lambda i: (0, i, 0, 0)),
            pl.BlockSpec((3, 3, C), lambda i: (0, 0, 0)),
            pl.BlockSpec((C, C), lambda i: (0, 0)),
            pl.BlockSpec((2, B, C), lambda i: (0, 0, 0)),
            pl.BlockSpec((B, H, 1, C), lambda i: (0, 0, 0, 0)),
            pl.BlockSpec((B, 1, W + 2, C), lambda i: (0, 0, 0, 0)),
            pl.BlockSpec((1, B, W, C), lambda i: (i, 0, 0, 0)),
        ],
        out_specs=pl.BlockSpec((B, BR, W, C), lambda i: (0, i, 0, 0)),
        scratch_shapes=[
            pltpu.VMEM((B, 1, W + 2, C), jnp.float32),
            pltpu.VMEM((B, BR + 2, 272, C), jnp.float32),
        ],
        compiler_params=_CP(dimension_semantics=("arbitrary",),
                            vmem_limit_bytes=56 * 1024 * 1024),
    )(x, k, Wp, stats, col_halo, row_halo, bots)


def kernel(x, k, Wp):
    stats, col_halo, row_halo = _prep(x)
    bot_idx = tuple(min((i + 1) * BR, H - 1) for i in range(NB))
    bots = jnp.moveaxis(x[:, bot_idx, :, :], 1, 0)
    return _main(x, k, Wp, stats, col_halo, row_halo, bots)
